# baseline (speedup 1.0000x reference)
import functools

import jax
import jax.numpy as jnp
from jax import lax
from jax.experimental import pallas as pl
from jax.experimental.pallas import tpu as pltpu
from jax.experimental.pallas import tpu_sc as plsc

N = 10000
E = 160000
F = 128
FH = 64
UNIT = 256
HID = 512
P = 18

NC = 2
NS = 16
LANES = 16

E_TOT = E + N
CHUNK = 64
CHUNKS_PER_TILE = 176
E_PER_TILE = CHUNKS_PER_TILE * CHUNK
E_PAD = E_PER_TILE * NS
EROWS = E_PAD // CHUNK
NPAD = 10240
NROWS_TILE = NPAD // NS
LAST_REAL = N - (NS - 1) * NROWS_TILE
DTAB = NPAD // CHUNK
NBUF = 8

_f32 = jnp.float32
_i32 = jnp.int32


def _zvec():
    return jnp.zeros((LANES,), _f32)


def _nrsqrt(x):
    xi = plsc.bitcast(x, _i32)
    y = plsc.bitcast(jnp.int32(0x5F3759DF) - lax.shift_right_logical(xi, 1), _f32)
    xh = 0.5 * jnp.maximum(x, 1e-12)
    for _ in range(3):
        y = y * (1.5 - xh * y * y)
    return jnp.where(x > 0, y, 0.0)


def _gnn_sc_body(xflat, src2, dst2, w2, zeros_hbm, s_out,
                 acc, srcv, dstv, wpv, ring, stage, semg, sems):
    c = lax.axis_index("c")
    s = lax.axis_index("s")
    base = s * CHUNKS_PER_TILE
    cbase = c * (P * N)
    myrow = s * NROWS_TILE

    pltpu.sync_copy(src2.at[pl.ds(base, CHUNKS_PER_TILE)], srcv)
    pltpu.sync_copy(dst2.at[pl.ds(base, CHUNKS_PER_TILE)], dstv)

    def zdeg(r, carry):
        for k in range(4):
            ring[r, pl.ds(k * LANES, LANES)] = _zvec()
        return carry

    lax.fori_loop(0, DTAB + 16, zdeg, 0)

    def ebatch(rr, carry):
        pltpu.sync_copy(w2.at[pl.ds(base + rr * 8, 8)], stage)
        for r8 in range(8):
            for k in range(4):
                sl = pl.ds(k * LANES, LANES)
                d16 = dstv[rr * 8 + r8, sl]
                plsc.addupdate_scatter(
                    ring,
                    [lax.shift_right_logical(d16, 6), lax.bitwise_and(d16, 63)],
                    stage[r8, sl],
                )
        return carry

    lax.fori_loop(0, CHUNKS_PER_TILE // 8, ebatch, 0)
    pltpu.sync_copy(ring.at[pl.ds(0, DTAB)],
                    s_out.at[pl.ds(cbase + s * DTAB, DTAB)])
    plsc.subcore_barrier()

    @pl.when(s < 10)
    def _reduce():
        for k in range(NS):
            for h in range(2):
                pltpu.sync_copy(
                    s_out.at[pl.ds(cbase + k * DTAB + s * 16 + h * 8, 8)], stage)
                for j in range(8):
                    for q in range(4):
                        sl = pl.ds(q * LANES, LANES)
                        ring[DTAB + h * 8 + j, sl] = (
                            ring[DTAB + h * 8 + j, sl] + stage[j, sl])
        pltpu.sync_copy(
            ring.at[pl.ds(DTAB, 16)],
            s_out.at[pl.ds(cbase + NS * DTAB + s * 16, 16)],
        )

    plsc.subcore_barrier()
    pltpu.sync_copy(s_out.at[pl.ds(cbase + NS * DTAB, DTAB)],
                    ring.at[pl.ds(0, DTAB)])

    def drow(r, carry):
        for k in range(4):
            sl = pl.ds(k * LANES, LANES)
            ring[r, sl] = _nrsqrt(ring[r, sl])
        return carry

    lax.fori_loop(0, DTAB, drow, 0)

    def wbatch(rr, carry):
        pltpu.sync_copy(w2.at[pl.ds(base + rr * 8, 8)], stage)
        for r8 in range(8):
            r = rr * 8 + r8
            for k in range(4):
                sl = pl.ds(k * LANES, LANES)
                s16 = srcv[r, sl]
                d16 = dstv[r, sl]
                ws = plsc.load_gather(
                    ring,
                    [lax.shift_right_logical(s16, 6), lax.bitwise_and(s16, 63)])
                wd = plsc.load_gather(
                    ring,
                    [lax.shift_right_logical(d16, 6), lax.bitwise_and(d16, 63)])
                wpv[pl.ds(r * CHUNK + k * LANES, LANES)] = ws * stage[r8, sl] * wd
        return carry

    lax.fori_loop(0, CHUNKS_PER_TILE // 8, wbatch, 0)

    def goff(r, carry):
        for k in range(4):
            sl = pl.ds(k * LANES, LANES)
            srcv[r, sl] = srcv[r, sl] * 2 + c
        return carry

    lax.fori_loop(0, CHUNKS_PER_TILE, goff, 0)

    def _gissue(ch):
        b = lax.rem(ch, NBUF)
        return pltpu.make_async_copy(
            xflat.at[srcv.at[ch]], ring.at[pl.ds(b * CHUNK, CHUNK)], semg)

    def _sissue(ch):
        b = lax.rem(ch, NBUF)
        return pltpu.make_async_copy(
            ring.at[pl.ds(b * CHUNK, CHUNK)], acc.at[dstv.at[ch]], sems)

    def period(t, carry):
        p = t

        @pl.when(s < NS - 1)
        def _z_full():
            pltpu.sync_copy(zeros_hbm, acc.at[pl.ds(myrow, NROWS_TILE)])

        @pl.when(s == NS - 1)
        def _z_last():
            pltpu.sync_copy(zeros_hbm.at[pl.ds(0, LAST_REAL)],
                            acc.at[pl.ds((NS - 1) * NROWS_TILE, LAST_REAL)])

        plsc.subcore_barrier()

        _gissue(jnp.int32(0)).start()
        _gissue(jnp.int32(1)).start()

        def chunk(ch, carry2):
            @pl.when(ch >= NBUF - 2)
            def _drain_s():
                _sissue(ch - (NBUF - 2)).wait()

            @pl.when(ch + 2 < CHUNKS_PER_TILE)
            def _prefetch():
                _gissue(ch + 2).start()

            _gissue(ch).wait()
            b = lax.rem(ch, NBUF)

            def sc16(e16, carry3):
                for i in range(LANES):
                    el = e16 * LANES + i
                    wb = plsc.load_gather(
                        wpv, [jnp.broadcast_to(ch * CHUNK + el, (LANES,)).astype(_i32)])
                    for j in range(4):
                        sl = pl.ds(j * LANES, LANES)
                        ring[b * CHUNK + el, sl] = ring[b * CHUNK + el, sl] * wb
                return carry3

            lax.fori_loop(0, CHUNK // LANES, sc16, 0)
            _sissue(ch).start(add=True)
            return carry2

        lax.fori_loop(0, CHUNKS_PER_TILE, chunk, 0)

        def sdrain(ch, carry2):
            _sissue(ch).wait()
            return carry2

        lax.fori_loop(CHUNKS_PER_TILE - (NBUF - 2), CHUNKS_PER_TILE, sdrain, 0)
        plsc.subcore_barrier()

        @pl.when(s < NS - 1)
        def _wb_full():
            pltpu.sync_copy(
                acc.at[pl.ds(myrow, NROWS_TILE)],
                s_out.at[pl.ds(cbase + p * N + myrow, NROWS_TILE)],
            )

        @pl.when(s == NS - 1)
        def _wb_last():
            pltpu.sync_copy(
                acc.at[pl.ds((NS - 1) * NROWS_TILE, LAST_REAL)],
                s_out.at[pl.ds(cbase + p * N + (NS - 1) * NROWS_TILE, LAST_REAL)],
            )

        def goff2(r, carry2):
            for k in range(4):
                sl = pl.ds(k * LANES, LANES)
                srcv[r, sl] = srcv[r, sl] + 2 * N
            return carry2

        lax.fori_loop(0, CHUNKS_PER_TILE, goff2, 0)
        return carry

    lax.fori_loop(0, P, period, 0)


def _dense_body(s_ref, wcz, wlz, bcz, blz, wch, wlh, bch, blh, att,
                w1, b1, w2, b2, w3, b3, w4, b4, out_ref):
    f32 = _f32
    wz = jnp.dot(wcz[...], wlz[...], preferred_element_type=f32)
    bz = jnp.dot(bcz[...], wlz[...], preferred_element_type=f32) + blz[...]
    wh = jnp.dot(wch[...], wlh[...], preferred_element_type=f32)
    bh = jnp.dot(bch[...], wlh[...], preferred_element_type=f32) + blh[...]
    probs = jax.nn.softmax(att[...], axis=-1)

    acc = jnp.zeros((s_ref.shape[2], UNIT), f32)
    for p in range(P):
        s0 = s_ref[0, p]
        s1 = s_ref[1, p]
        gz = (jnp.dot(s0, wz[:FH], preferred_element_type=f32)
              + jnp.dot(s1, wz[FH:], preferred_element_type=f32) + bz)
        gh = (jnp.dot(s0, wh[:FH], preferred_element_type=f32)
              + jnp.dot(s1, wh[FH:], preferred_element_type=f32) + bh)
        acc = acc + probs[0, p] * (1.0 - jax.nn.sigmoid(gz)) * jnp.tanh(gh)

    h = jax.nn.relu(acc)
    h = jax.nn.relu(jnp.dot(h, w1[...], preferred_element_type=f32) + b1[...])
    h = jax.nn.relu(jnp.dot(h, w2[...], preferred_element_type=f32) + b2[...])
    h = jax.nn.relu(jnp.dot(h, w3[...], preferred_element_type=f32) + b3[...])
    out_ref[...] = jnp.dot(h, w4[...], preferred_element_type=f32) + b4[...]


def _full(shape):
    return pl.BlockSpec(shape, lambda i: tuple(0 for _ in shape))


def kernel(x, edge_index, edge_weight, attention, Wc_z, bc_z, Wl_z, bl_z,
           Wc_r, bc_r, Wl_r, bl_r, Wc_h, bc_h, Wl_h, bl_h,
           W1, b1, W2, b2, W3, b3, W4, b4):
    xflat = jnp.transpose(x, (2, 0, 1)).reshape(P * N * 2, FH)
    loop = jnp.arange(N, dtype=_i32)
    pad = E_PAD - E_TOT
    src_all = jnp.concatenate([edge_index[0], loop, jnp.zeros((pad,), _i32)])
    dst_all = jnp.concatenate([edge_index[1], loop, jnp.zeros((pad,), _i32)])
    w_all = jnp.concatenate([edge_weight, jnp.ones((N,), _f32), jnp.zeros((pad,), _f32)])
    src2 = src_all.reshape(EROWS, CHUNK)
    dst2 = dst_all.reshape(EROWS, CHUNK)
    w2 = w_all.reshape(EROWS, CHUNK)

    mesh = plsc.VectorSubcoreMesh(core_axis_name="c", subcore_axis_name="s")

    sc_kernel = functools.partial(
        pl.kernel,
        out_type=jax.ShapeDtypeStruct((NC * P * N, FH), _f32),
        mesh=mesh,
        compiler_params=pltpu.CompilerParams(
            needs_layout_passes=False, use_tc_tiling_on_sc=False),
        scratch_types=[
            pltpu.VMEM_SHARED((N, FH), _f32),
            pltpu.VMEM((CHUNKS_PER_TILE, CHUNK), _i32),
            pltpu.VMEM((CHUNKS_PER_TILE, CHUNK), _i32),
            pltpu.VMEM((E_PER_TILE,), _f32),
            pltpu.VMEM((NBUF * CHUNK, FH), _f32),
            pltpu.VMEM((8, CHUNK), _f32),
            pltpu.SemaphoreType.DMA,
            pltpu.SemaphoreType.DMA,
        ],
    )(_gnn_sc_body)
    zeros_hbm = jnp.zeros((NROWS_TILE, FH), _f32)
    s4 = sc_kernel(xflat, src2, dst2, w2, zeros_hbm).reshape(NC, P, N, FH)

    tile = 1000
    wlz_top = Wl_z[:UNIT]
    wlh_top = Wl_h[:UNIT]
    out = pl.pallas_call(
        _dense_body,
        grid=(N // tile,),
        in_specs=[
            pl.BlockSpec((NC, P, tile, FH), lambda i: (0, 0, i, 0)),
            _full((F, UNIT)), _full((UNIT, UNIT)), _full((1, UNIT)), _full((1, UNIT)),
            _full((F, UNIT)), _full((UNIT, UNIT)), _full((1, UNIT)), _full((1, UNIT)),
            _full((1, P)),
            _full((UNIT, HID)), _full((1, HID)),
            _full((HID, HID)), _full((1, HID)),
            _full((HID, HID)), _full((1, HID)),
            _full((HID, P)), _full((1, P)),
        ],
        out_specs=pl.BlockSpec((tile, P), lambda i: (i, 0)),
        out_shape=jax.ShapeDtypeStruct((N, P), _f32),
    )(s4, Wc_z, wlz_top, bc_z.reshape(1, UNIT), bl_z.reshape(1, UNIT),
      Wc_h, wlh_top, bc_h.reshape(1, UNIT), bl_h.reshape(1, UNIT),
      attention.reshape(1, P),
      W1, b1.reshape(1, HID), W2, b2.reshape(1, HID),
      W3, b3.reshape(1, HID), W4, b4.reshape(1, P))
    return out


# speedup vs baseline: 1.1658x; 1.1658x over previous
import functools

import jax
import jax.numpy as jnp
from jax import lax
from jax.experimental import pallas as pl
from jax.experimental.pallas import tpu as pltpu
from jax.experimental.pallas import tpu_sc as plsc

N = 10000
E = 160000
F = 128
UNIT = 256
HID = 512
P = 18

NC = 2
NS = 16
LANES = 16

E_TOT = E + N
CHUNK = 128
CHUNKS_PER_TILE = 88
E_PER_TILE = CHUNKS_PER_TILE * CHUNK
E_PAD = E_PER_TILE * NS
EROWS = E_PAD // CHUNK
PERIODS_PER_CORE = P // NC
NPAD = 10240
N_PER_TILE = NPAD // NS
LAST_REAL = N - 15 * N_PER_TILE
ZROWS = 80

_f32 = jnp.float32
_i32 = jnp.int32


def _zvec():
    return jnp.zeros((LANES,), _f32)


def _nrsqrt(x):
    xi = plsc.bitcast(x, _i32)
    y = plsc.bitcast(jnp.int32(0x5F3759DF) - lax.shift_right_logical(xi, 1), _f32)
    xh = 0.5 * jnp.maximum(x, 1e-12)
    for _ in range(3):
        y = y * (1.5 - xh * y * y)
    return jnp.where(x > 0, y, 0.0)


def _gnn_sc_body(xflat, src2, dst2, w2, zeros_hbm, s_out,
                 acc, gidxv, dstv, wpv, rowbuf, buf4, sem):
    c = lax.axis_index("c")
    s = lax.axis_index("s")
    base = s * CHUNKS_PER_TILE
    DROWS = NPAD // CHUNK
    cbase = c * PERIODS_PER_CORE * N
    nrows = NPAD // NS
    myrow = s * nrows

    pltpu.sync_copy(src2.at[pl.ds(base, CHUNKS_PER_TILE)], gidxv)
    pltpu.sync_copy(dst2.at[pl.ds(base, CHUNKS_PER_TILE)], dstv)

    def zdeg(r, carry):
        for k in range(8):
            rowbuf[r, pl.ds(k * LANES, LANES)] = _zvec()
        return carry

    lax.fori_loop(0, DROWS, zdeg, 0)

    def ebatch(rr, carry):
        pltpu.sync_copy(w2.at[pl.ds(base + rr * 4, 4)], buf4)
        for r4 in range(4):
            for k in range(8):
                sl = pl.ds(k * LANES, LANES)
                d16 = dstv[rr * 4 + r4, sl]
                plsc.addupdate_scatter(
                    rowbuf,
                    [lax.shift_right_logical(d16, 7), lax.bitwise_and(d16, 127)],
                    buf4[r4, sl],
                )
        return carry

    lax.fori_loop(0, CHUNKS_PER_TILE // 4, ebatch, 0)
    pltpu.sync_copy(rowbuf.at[pl.ds(0, DROWS)],
                    s_out.at[pl.ds(cbase + s * DROWS, DROWS)])
    plsc.subcore_barrier()

    @pl.when(s < 10)
    def _reduce():
        for j in range(8):
            for k in range(8):
                rowbuf[DROWS + j, pl.ds(k * LANES, LANES)] = _zvec()
        for k in range(NS):
            for h in range(2):
                pltpu.sync_copy(
                    s_out.at[pl.ds(cbase + k * DROWS + s * 8 + h * 4, 4)], buf4)
                for j in range(4):
                    for q in range(8):
                        sl = pl.ds(q * LANES, LANES)
                        rowbuf[DROWS + h * 4 + j, sl] = (
                            rowbuf[DROWS + h * 4 + j, sl] + buf4[j, sl])
        pltpu.sync_copy(
            rowbuf.at[pl.ds(DROWS, 8)],
            s_out.at[pl.ds(cbase + NS * DROWS + s * 8, 8)],
        )

    plsc.subcore_barrier()
    pltpu.sync_copy(s_out.at[pl.ds(cbase + NS * DROWS, DROWS)],
                    rowbuf.at[pl.ds(0, DROWS)])

    def drow(r, carry):
        for k in range(8):
            sl = pl.ds(k * LANES, LANES)
            rowbuf[r, sl] = _nrsqrt(rowbuf[r, sl])
        return carry

    lax.fori_loop(0, DROWS, drow, 0)

    def wbatch(rr, carry):
        pltpu.sync_copy(w2.at[pl.ds(base + rr * 4, 4)], buf4)
        for r4 in range(4):
            r = rr * 4 + r4
            for k in range(8):
                sl = pl.ds(k * LANES, LANES)
                s16 = gidxv[r, sl]
                d16 = dstv[r, sl]
                ws = plsc.load_gather(
                    rowbuf,
                    [lax.shift_right_logical(s16, 7), lax.bitwise_and(s16, 127)])
                wd = plsc.load_gather(
                    rowbuf,
                    [lax.shift_right_logical(d16, 7), lax.bitwise_and(d16, 127)])
                wpv[pl.ds(r * CHUNK + k * LANES, LANES)] = ws * buf4[r4, sl] * wd
        return carry

    lax.fori_loop(0, CHUNKS_PER_TILE // 4, wbatch, 0)

    def goff(r, carry):
        for k in range(8):
            sl = pl.ds(k * LANES, LANES)
            gidxv[r, sl] = gidxv[r, sl] + c * (PERIODS_PER_CORE * N)
        return carry

    lax.fori_loop(0, CHUNKS_PER_TILE, goff, 0)

    def period(t, carry):
        p = c * PERIODS_PER_CORE + t

        @pl.when(s < NS - 1)
        def _z_full():
            pltpu.sync_copy(zeros_hbm, acc.at[pl.ds(myrow, nrows)])

        @pl.when(s == NS - 1)
        def _z_last():
            pltpu.sync_copy(zeros_hbm.at[pl.ds(0, LAST_REAL)],
                            acc.at[pl.ds((NS - 1) * nrows, LAST_REAL)])

        plsc.subcore_barrier()

        def chunk(ch, carry2):
            pltpu.async_copy(xflat.at[gidxv.at[ch]], rowbuf, sem).wait()

            def sc16(e16, carry3):
                for i in range(LANES):
                    el = e16 * LANES + i
                    wb = plsc.load_gather(
                        wpv, [jnp.broadcast_to(ch * CHUNK + el, (LANES,)).astype(_i32)])
                    for j in range(8):
                        sl = pl.ds(j * LANES, LANES)
                        rowbuf[el, sl] = rowbuf[el, sl] * wb
                return carry3

            lax.fori_loop(0, CHUNK // LANES, sc16, 0)
            pltpu.sync_copy(rowbuf, acc.at[dstv.at[ch]], add=True)
            return carry2

        lax.fori_loop(0, CHUNKS_PER_TILE, chunk, 0)
        plsc.subcore_barrier()

        @pl.when(s < NS - 1)
        def _wb_full():
            pltpu.sync_copy(
                acc.at[pl.ds(myrow, nrows)],
                s_out.at[pl.ds(p * N + myrow, nrows)],
            )

        @pl.when(s == NS - 1)
        def _wb_last():
            pltpu.sync_copy(
                acc.at[pl.ds((NS - 1) * nrows, LAST_REAL)],
                s_out.at[pl.ds(p * N + (NS - 1) * nrows, LAST_REAL)],
            )

        def goff2(r, carry2):
            for k in range(8):
                sl = pl.ds(k * LANES, LANES)
                gidxv[r, sl] = gidxv[r, sl] + N
            return carry2

        lax.fori_loop(0, CHUNKS_PER_TILE, goff2, 0)
        return carry

    lax.fori_loop(0, PERIODS_PER_CORE, period, 0)


def _dense_body(s_ref, wcz, wlz, bcz, blz, wch, wlh, bch, blh, att,
                w1, b1, w2, b2, w3, b3, w4, b4, out_ref):
    f32 = _f32
    wz = jnp.dot(wcz[...], wlz[...], preferred_element_type=f32)
    bz = jnp.dot(bcz[...], wlz[...], preferred_element_type=f32) + blz[...]
    wh = jnp.dot(wch[...], wlh[...], preferred_element_type=f32)
    bh = jnp.dot(bch[...], wlh[...], preferred_element_type=f32) + blh[...]
    probs = jax.nn.softmax(att[...], axis=-1)

    acc = jnp.zeros((s_ref.shape[1], UNIT), f32)
    for p in range(P):
        sp = s_ref[p]
        gz = jnp.dot(sp, wz, preferred_element_type=f32) + bz
        gh = jnp.dot(sp, wh, preferred_element_type=f32) + bh
        acc = acc + probs[0, p] * (1.0 - jax.nn.sigmoid(gz)) * jnp.tanh(gh)

    h = jax.nn.relu(acc)
    h = jax.nn.relu(jnp.dot(h, w1[...], preferred_element_type=f32) + b1[...])
    h = jax.nn.relu(jnp.dot(h, w2[...], preferred_element_type=f32) + b2[...])
    h = jax.nn.relu(jnp.dot(h, w3[...], preferred_element_type=f32) + b3[...])
    out_ref[...] = jnp.dot(h, w4[...], preferred_element_type=f32) + b4[...]


def _full(shape):
    return pl.BlockSpec(shape, lambda i: tuple(0 for _ in shape))


def kernel(x, edge_index, edge_weight, attention, Wc_z, bc_z, Wl_z, bl_z,
           Wc_r, bc_r, Wl_r, bl_r, Wc_h, bc_h, Wl_h, bl_h,
           W1, b1, W2, b2, W3, b3, W4, b4):
    xflat = jnp.transpose(x, (2, 0, 1)).reshape(P * N, F)
    loop = jnp.arange(N, dtype=_i32)
    pad = E_PAD - E_TOT
    src_all = jnp.concatenate([edge_index[0], loop, jnp.zeros((pad,), _i32)])
    dst_all = jnp.concatenate([edge_index[1], loop, jnp.zeros((pad,), _i32)])
    w_all = jnp.concatenate([edge_weight, jnp.ones((N,), _f32), jnp.zeros((pad,), _f32)])
    src2 = src_all.reshape(EROWS, CHUNK)
    dst2 = dst_all.reshape(EROWS, CHUNK)
    w2 = w_all.reshape(EROWS, CHUNK)

    mesh = plsc.VectorSubcoreMesh(core_axis_name="c", subcore_axis_name="s")

    sc_kernel = functools.partial(
        pl.kernel,
        out_type=jax.ShapeDtypeStruct((P * N, F), _f32),
        mesh=mesh,
        compiler_params=pltpu.CompilerParams(
            needs_layout_passes=False, use_tc_tiling_on_sc=False),
        scratch_types=[
            pltpu.VMEM_SHARED((N, F), _f32),
            pltpu.VMEM((CHUNKS_PER_TILE, CHUNK), _i32),
            pltpu.VMEM((CHUNKS_PER_TILE, CHUNK), _i32),
            pltpu.VMEM((E_PER_TILE,), _f32),
            pltpu.VMEM((CHUNK, F), _f32),
            pltpu.VMEM((4, CHUNK), _f32),
            pltpu.SemaphoreType.DMA,
        ],
    )(_gnn_sc_body)
    zeros_hbm = jnp.zeros((NPAD // NS, F), _f32)
    s3 = sc_kernel(xflat, src2, dst2, w2, zeros_hbm).reshape(P, N, F)

    tile = 1000
    wlz_top = Wl_z[:UNIT]
    wlh_top = Wl_h[:UNIT]
    out = pl.pallas_call(
        _dense_body,
        grid=(N // tile,),
        in_specs=[
            pl.BlockSpec((P, tile, F), lambda i: (0, i, 0)),
            _full((F, UNIT)), _full((UNIT, UNIT)), _full((1, UNIT)), _full((1, UNIT)),
            _full((F, UNIT)), _full((UNIT, UNIT)), _full((1, UNIT)), _full((1, UNIT)),
            _full((1, P)),
            _full((UNIT, HID)), _full((1, HID)),
            _full((HID, HID)), _full((1, HID)),
            _full((HID, HID)), _full((1, HID)),
            _full((HID, P)), _full((1, P)),
        ],
        out_specs=pl.BlockSpec((tile, P), lambda i: (i, 0)),
        out_shape=jax.ShapeDtypeStruct((N, P), _f32),
    )(s3, Wc_z, wlz_top, bc_z.reshape(1, UNIT), bl_z.reshape(1, UNIT),
      Wc_h, wlh_top, bc_h.reshape(1, UNIT), bl_h.reshape(1, UNIT),
      attention.reshape(1, P),
      W1, b1.reshape(1, HID), W2, b2.reshape(1, HID),
      W3, b3.reshape(1, HID), W4, b4.reshape(1, P))
    return out
